# Initial kernel scaffold; baseline (speedup 1.0000x reference)
import jax
import jax.numpy as jnp
from jax.experimental import pallas as pl


def kernel(x, edge_index, W, b):
    raise NotImplementedError("write your pallas kernel here")


# speedup vs baseline: 6.7469x; 6.7469x over previous
import functools

import jax
import jax.numpy as jnp
from jax import lax
from jax.experimental import pallas as pl
from jax.experimental.pallas import tpu as pltpu
from jax.experimental.pallas import tpu_sc as plsc

N_NODES = 10000
IN_DIM = 128
OUT_DIM = 128
N_EDGES = 320000

CHUNK = 128
N_CHUNKS = N_EDGES // CHUNK
NW = 32
N_PIECES = N_NODES // CHUNK
TAIL_BASE = N_PIECES * CHUNK
TAIL_ROWS = N_NODES - TAIL_BASE

_FULL, _REM = divmod(N_CHUNKS, NW)

_sc_mesh = plsc.VectorSubcoreMesh(core_axis_name="c", subcore_axis_name="s")


@functools.partial(
    pl.kernel,
    out_type=jax.ShapeDtypeStruct((2 * N_NODES, IN_DIM), jnp.float32),
    mesh=_sc_mesh,
    scratch_types=[
        pltpu.VMEM((CHUNK,), jnp.int32),
        pltpu.VMEM((CHUNK,), jnp.int32),
        pltpu.VMEM((CHUNK, IN_DIM), jnp.float32),
        pltpu.VMEM_SHARED((N_NODES, IN_DIM), jnp.float32),
        pltpu.SemaphoreType.DMA,
    ],
)
def _sc_aggregate(src_hbm, dst_hbm, x_hbm, out_hbm, sidx, didx, rows, acc, sem):
    core = lax.axis_index("c")
    sub = lax.axis_index("s")
    wid = core * 16 + sub

    zero16 = jnp.zeros((16,), jnp.float32)

    def zbody(i, carry):
        r = i // (IN_DIM // 16)
        j = i - r * (IN_DIM // 16)
        rows[r, pl.ds(j * 16, 16)] = zero16
        return carry

    lax.fori_loop(0, CHUNK * (IN_DIM // 16), zbody, 0)
    npiece = jnp.where(sub < 14, 5, 4)

    def zpiece(q, carry):
        pltpu.sync_copy(rows, acc.at[pl.ds((sub + q * 16) * CHUNK, CHUNK)])
        return carry

    lax.fori_loop(0, npiece, zpiece, 0)

    @pl.when(sub == 15)
    def _zero_tail():
        pltpu.sync_copy(rows.at[pl.ds(0, TAIL_ROWS)], acc.at[pl.ds(TAIL_BASE, TAIL_ROWS)])

    plsc.subcore_barrier()

    nchunk = jnp.where(wid < _REM, _FULL + 1, _FULL)

    def body(k, carry):
        base = (wid + k * NW) * CHUNK
        pltpu.sync_copy(src_hbm.at[pl.ds(base, CHUNK)], sidx)
        pltpu.sync_copy(dst_hbm.at[pl.ds(base, CHUNK)], didx)
        pltpu.async_copy(x_hbm.at[sidx], rows, sem).wait()
        pltpu.sync_copy(rows, acc.at[didx], add=True)
        return carry

    lax.fori_loop(0, nchunk, body, 0)
    plsc.subcore_barrier()

    out0 = core * N_NODES

    def dpiece(q, carry):
        base = (sub + q * 16) * CHUNK
        pltpu.sync_copy(acc.at[pl.ds(base, CHUNK)], rows)
        pltpu.sync_copy(rows, out_hbm.at[pl.ds(out0 + base, CHUNK)])
        return carry

    lax.fori_loop(0, npiece, dpiece, 0)

    @pl.when(sub == 15)
    def _drain_tail():
        pltpu.sync_copy(acc.at[pl.ds(TAIL_BASE, TAIL_ROWS)], rows.at[pl.ds(0, TAIL_ROWS)])
        pltpu.sync_copy(rows.at[pl.ds(0, TAIL_ROWS)],
                        out_hbm.at[pl.ds(out0 + TAIL_BASE, TAIL_ROWS)])


_M_BLK = 2000


def _tc_body(x_ref, a0_ref, a1_ref, w_ref, b_ref, o_ref):
    h = x_ref[...] + a0_ref[...] + a1_ref[...]
    o_ref[...] = (
        jnp.dot(h, w_ref[...], preferred_element_type=jnp.float32) + b_ref[...]
    )


def _tc_mlp(x, agg2, W, b2):
    n_blk = N_NODES // _M_BLK
    return pl.pallas_call(
        _tc_body,
        grid=(n_blk,),
        in_specs=[
            pl.BlockSpec((_M_BLK, IN_DIM), lambda i: (i, 0)),
            pl.BlockSpec((_M_BLK, IN_DIM), lambda i: (i, 0)),
            pl.BlockSpec((_M_BLK, IN_DIM), lambda i: (i + n_blk, 0)),
            pl.BlockSpec((IN_DIM, OUT_DIM), lambda i: (0, 0)),
            pl.BlockSpec((1, OUT_DIM), lambda i: (0, 0)),
        ],
        out_specs=pl.BlockSpec((_M_BLK, OUT_DIM), lambda i: (i, 0)),
        out_shape=jax.ShapeDtypeStruct((N_NODES, OUT_DIM), jnp.float32),
    )(x, agg2, agg2, W, b2)


def kernel(x, edge_index, W, b):
    ei = edge_index.astype(jnp.int32)
    agg2 = _sc_aggregate(ei[0], ei[1], x)
    return _tc_mlp(x, agg2, W, b.reshape(1, OUT_DIM))


# speedup vs baseline: 10.2597x; 1.5207x over previous
import functools

import jax
import jax.numpy as jnp
from jax import lax
from jax.experimental import pallas as pl
from jax.experimental.pallas import tpu as pltpu
from jax.experimental.pallas import tpu_sc as plsc

N_NODES = 10000
IN_DIM = 128
OUT_DIM = 128
N_EDGES = 320000

CHUNK = 128
N_CHUNKS = N_EDGES // CHUNK
NW = 32
N_PIECES = N_NODES // CHUNK
TAIL_BASE = N_PIECES * CHUNK
TAIL_ROWS = N_NODES - TAIL_BASE

_FULL, _REM = divmod(N_CHUNKS, NW)

_sc_mesh = plsc.VectorSubcoreMesh(core_axis_name="c", subcore_axis_name="s")


@functools.partial(
    pl.kernel,
    out_type=jax.ShapeDtypeStruct((2 * N_NODES, IN_DIM), jnp.float32),
    mesh=_sc_mesh,
    scratch_types=[
        pltpu.VMEM((CHUNK,), jnp.int32),
        pltpu.VMEM((CHUNK,), jnp.int32),
        pltpu.VMEM((CHUNK, IN_DIM), jnp.float32),
        pltpu.SemaphoreType.DMA,
        pltpu.VMEM((CHUNK,), jnp.int32),
        pltpu.VMEM((CHUNK,), jnp.int32),
        pltpu.VMEM((CHUNK, IN_DIM), jnp.float32),
        pltpu.SemaphoreType.DMA,
        pltpu.VMEM_SHARED((N_NODES, IN_DIM), jnp.float32),
    ],
)
def _sc_aggregate(src_hbm, dst_hbm, x_hbm, out_hbm,
                  sidxA, didxA, rowsA, gsemA, sidxB, didxB, rowsB, gsemB, acc):
    core = lax.axis_index("c")
    sub = lax.axis_index("s")
    wid = core * 16 + sub

    zero16 = jnp.zeros((16,), jnp.float32)

    def zbody(i, carry):
        r = i // (IN_DIM // 16)
        j = i - r * (IN_DIM // 16)
        rowsA[r, pl.ds(j * 16, 16)] = zero16
        return carry

    lax.fori_loop(0, CHUNK * (IN_DIM // 16), zbody, 0)
    npiece = jnp.where(sub < 14, 5, 4)

    def zpiece(q, carry):
        pltpu.sync_copy(rowsA, acc.at[pl.ds((sub + q * 16) * CHUNK, CHUNK)])
        return carry

    lax.fori_loop(0, npiece, zpiece, 0)

    @pl.when(sub == 15)
    def _zero_tail():
        pltpu.sync_copy(rowsA.at[pl.ds(0, TAIL_ROWS)], acc.at[pl.ds(TAIL_BASE, TAIL_ROWS)])

    plsc.subcore_barrier()

    nchunk = jnp.where(wid < _REM, _FULL + 1, _FULL)

    def fetch_and_gather(k, sidx, didx, rws, gsem):
        base = (wid + k * NW) * CHUNK
        pltpu.sync_copy(src_hbm.at[pl.ds(base, CHUNK)], sidx)
        pltpu.sync_copy(dst_hbm.at[pl.ds(base, CHUNK)], didx)
        pltpu.async_copy(x_hbm.at[sidx], rws, gsem)

    fetch_and_gather(0, sidxA, didxA, rowsA, gsemA)
    fetch_and_gather(1, sidxB, didxB, rowsB, gsemB)

    def body(j, carry):
        k0 = 2 * j
        pltpu.make_async_copy(x_hbm.at[sidxA], rowsA, gsemA).wait()
        pltpu.sync_copy(rowsA, acc.at[didxA], add=True)

        @pl.when(k0 + 2 < nchunk)
        def _prefA():
            fetch_and_gather(k0 + 2, sidxA, didxA, rowsA, gsemA)

        @pl.when(k0 + 1 < nchunk)
        def _doB():
            pltpu.make_async_copy(x_hbm.at[sidxB], rowsB, gsemB).wait()
            pltpu.sync_copy(rowsB, acc.at[didxB], add=True)

            @pl.when(k0 + 3 < nchunk)
            def _prefB():
                fetch_and_gather(k0 + 3, sidxB, didxB, rowsB, gsemB)

        return carry

    lax.fori_loop(0, (nchunk + 1) // 2, body, 0)
    plsc.subcore_barrier()

    out0 = core * N_NODES

    def dpiece(q, carry):
        base = (sub + q * 16) * CHUNK
        pltpu.sync_copy(acc.at[pl.ds(base, CHUNK)], rowsA)
        pltpu.sync_copy(rowsA, out_hbm.at[pl.ds(out0 + base, CHUNK)])
        return carry

    lax.fori_loop(0, npiece, dpiece, 0)

    @pl.when(sub == 15)
    def _drain_tail():
        pltpu.sync_copy(acc.at[pl.ds(TAIL_BASE, TAIL_ROWS)], rowsA.at[pl.ds(0, TAIL_ROWS)])
        pltpu.sync_copy(rowsA.at[pl.ds(0, TAIL_ROWS)],
                        out_hbm.at[pl.ds(out0 + TAIL_BASE, TAIL_ROWS)])


_M_BLK = 2000


def _tc_body(x_ref, a0_ref, a1_ref, w_ref, b_ref, o_ref):
    h = x_ref[...] + a0_ref[...] + a1_ref[...]
    o_ref[...] = (
        jnp.dot(h, w_ref[...], preferred_element_type=jnp.float32) + b_ref[...]
    )


def _tc_mlp(x, agg2, W, b2):
    n_blk = N_NODES // _M_BLK
    return pl.pallas_call(
        _tc_body,
        grid=(n_blk,),
        in_specs=[
            pl.BlockSpec((_M_BLK, IN_DIM), lambda i: (i, 0)),
            pl.BlockSpec((_M_BLK, IN_DIM), lambda i: (i, 0)),
            pl.BlockSpec((_M_BLK, IN_DIM), lambda i: (i + n_blk, 0)),
            pl.BlockSpec((IN_DIM, OUT_DIM), lambda i: (0, 0)),
            pl.BlockSpec((1, OUT_DIM), lambda i: (0, 0)),
        ],
        out_specs=pl.BlockSpec((_M_BLK, OUT_DIM), lambda i: (i, 0)),
        out_shape=jax.ShapeDtypeStruct((N_NODES, OUT_DIM), jnp.float32),
    )(x, agg2, agg2, W, b2)


def kernel(x, edge_index, W, b):
    ei = edge_index.astype(jnp.int32)
    agg2 = _sc_aggregate(ei[0], ei[1], x)
    return _tc_mlp(x, agg2, W, b.reshape(1, OUT_DIM))


# speedup vs baseline: 12.7805x; 1.2457x over previous
import functools

import jax
import jax.numpy as jnp
from jax import lax
from jax.experimental import pallas as pl
from jax.experimental.pallas import tpu as pltpu
from jax.experimental.pallas import tpu_sc as plsc

N_NODES = 10000
IN_DIM = 128
OUT_DIM = 128
N_EDGES = 320000

CHUNK = 128
NW = 32
CHUNKS_PER_W = 80
N_CHUNKS_P = NW * CHUNKS_PER_W
PAD_EDGES = N_CHUNKS_P * CHUNK - N_EDGES
GRP = 40
N_TRASH = 48
ACC_ROWS = N_NODES + N_TRASH

N_PIECES = N_NODES // CHUNK
TAIL_BASE = N_PIECES * CHUNK
ZTAIL_ROWS = ACC_ROWS - TAIL_BASE
DTAIL_ROWS = N_NODES - TAIL_BASE

_sc_mesh = plsc.VectorSubcoreMesh(core_axis_name="c", subcore_axis_name="s")


@functools.partial(
    pl.kernel,
    out_type=jax.ShapeDtypeStruct((2 * N_NODES, IN_DIM), jnp.float32),
    mesh=_sc_mesh,
    scratch_types=[
        pltpu.VMEM((GRP, CHUNK), jnp.int32),
        pltpu.VMEM((GRP, CHUNK), jnp.int32),
        pltpu.VMEM((CHUNK, IN_DIM), jnp.float32),
        pltpu.SemaphoreType.DMA,
        pltpu.VMEM((CHUNK, IN_DIM), jnp.float32),
        pltpu.SemaphoreType.DMA,
        pltpu.VMEM_SHARED((ACC_ROWS, IN_DIM), jnp.float32),
    ],
)
def _sc_aggregate(src_hbm, dst_hbm, x_hbm, out_hbm,
                  sgrp, dgrp, rowsA, gsemA, rowsB, gsemB, acc):
    core = lax.axis_index("c")
    sub = lax.axis_index("s")
    wid = core * 16 + sub

    zero16 = jnp.zeros((16,), jnp.float32)

    def zbody(i, carry):
        r = i // (IN_DIM // 16)
        j = i - r * (IN_DIM // 16)
        rowsA[r, pl.ds(j * 16, 16)] = zero16
        return carry

    lax.fori_loop(0, CHUNK * (IN_DIM // 16), zbody, 0)
    npiece = jnp.where(sub < 14, 5, 4)

    def zpiece(q, carry):
        pltpu.sync_copy(rowsA, acc.at[pl.ds((sub + q * 16) * CHUNK, CHUNK)])
        return carry

    lax.fori_loop(0, npiece, zpiece, 0)

    @pl.when(sub == 15)
    def _zero_tail():
        pltpu.sync_copy(rowsA.at[pl.ds(0, ZTAIL_ROWS)],
                        acc.at[pl.ds(TAIL_BASE, ZTAIL_ROWS)])

    plsc.subcore_barrier()

    def gather(i, rws, gsem):
        pltpu.async_copy(x_hbm.at[sgrp.at[i]], rws, gsem)

    def scatter(i, rws):
        pltpu.sync_copy(rws, acc.at[dgrp.at[i]], add=True)

    for g in range(CHUNKS_PER_W // GRP):
        gbase = wid * CHUNKS_PER_W + g * GRP
        pltpu.sync_copy(src_hbm.at[pl.ds(gbase, GRP)], sgrp)
        pltpu.sync_copy(dst_hbm.at[pl.ds(gbase, GRP)], dgrp)

        gather(0, rowsA, gsemA)
        gather(1, rowsB, gsemB)

        def body(j, carry):
            k0 = 2 * j
            pltpu.make_async_copy(x_hbm.at[sgrp.at[k0]], rowsA, gsemA).wait()
            scatter(k0, rowsA)
            gather(k0 + 2, rowsA, gsemA)
            pltpu.make_async_copy(x_hbm.at[sgrp.at[k0 + 1]], rowsB, gsemB).wait()
            scatter(k0 + 1, rowsB)
            gather(k0 + 3, rowsB, gsemB)
            return carry

        lax.fori_loop(0, GRP // 2 - 1, body, 0)
        pltpu.make_async_copy(x_hbm.at[sgrp.at[GRP - 2]], rowsA, gsemA).wait()
        scatter(GRP - 2, rowsA)
        pltpu.make_async_copy(x_hbm.at[sgrp.at[GRP - 1]], rowsB, gsemB).wait()
        scatter(GRP - 1, rowsB)

    plsc.subcore_barrier()

    out0 = core * N_NODES

    def dpiece(q, carry):
        base = (sub + q * 16) * CHUNK
        pltpu.sync_copy(acc.at[pl.ds(base, CHUNK)], rowsA)
        pltpu.sync_copy(rowsA, out_hbm.at[pl.ds(out0 + base, CHUNK)])
        return carry

    lax.fori_loop(0, npiece, dpiece, 0)

    @pl.when(sub == 15)
    def _drain_tail():
        pltpu.sync_copy(acc.at[pl.ds(TAIL_BASE, DTAIL_ROWS)],
                        rowsA.at[pl.ds(0, DTAIL_ROWS)])
        pltpu.sync_copy(rowsA.at[pl.ds(0, DTAIL_ROWS)],
                        out_hbm.at[pl.ds(out0 + TAIL_BASE, DTAIL_ROWS)])


_M_BLK = 2000


def _tc_body(x_ref, a0_ref, a1_ref, w_ref, b_ref, o_ref):
    h = x_ref[...] + a0_ref[...] + a1_ref[...]
    o_ref[...] = (
        jnp.dot(h, w_ref[...], preferred_element_type=jnp.float32) + b_ref[...]
    )


def _tc_mlp(x, agg2, W, b2):
    n_blk = N_NODES // _M_BLK
    return pl.pallas_call(
        _tc_body,
        grid=(n_blk,),
        in_specs=[
            pl.BlockSpec((_M_BLK, IN_DIM), lambda i: (i, 0)),
            pl.BlockSpec((_M_BLK, IN_DIM), lambda i: (i, 0)),
            pl.BlockSpec((_M_BLK, IN_DIM), lambda i: (i + n_blk, 0)),
            pl.BlockSpec((IN_DIM, OUT_DIM), lambda i: (0, 0)),
            pl.BlockSpec((1, OUT_DIM), lambda i: (0, 0)),
        ],
        out_specs=pl.BlockSpec((_M_BLK, OUT_DIM), lambda i: (i, 0)),
        out_shape=jax.ShapeDtypeStruct((N_NODES, OUT_DIM), jnp.float32),
    )(x, agg2, agg2, W, b2)


def kernel(x, edge_index, W, b):
    ei = edge_index.astype(jnp.int32)
    pad_iota = jnp.arange(PAD_EDGES, dtype=jnp.int32)
    pad_src = (pad_iota * 131) % N_NODES
    pad_dst = N_NODES + pad_iota % N_TRASH
    src2 = jnp.concatenate([ei[0], pad_src]).reshape(N_CHUNKS_P, CHUNK)
    dst2 = jnp.concatenate([ei[1], pad_dst]).reshape(N_CHUNKS_P, CHUNK)
    agg2 = _sc_aggregate(src2, dst2, x)
    return _tc_mlp(x, agg2, W, b.reshape(1, OUT_DIM))
